# baseline (speedup 1.0000x reference)
import functools

import jax
import jax.numpy as jnp
from jax import lax
from jax.experimental import pallas as pl
from jax.experimental.pallas import tpu as pltpu
from jax.experimental.pallas import tpu_sc as plsc

_NW = 32
_CHUNK = 128
_LANES = 16
_BV = 4096
_BVO = 4096
_BVF = 2048
_BVT = 12800


def _pad_table_tensorcore(embT, V, E):
    n_v = -(-V // _BVT)

    def body(t_ref, o_ref):
        o_ref[:, pl.ds(0, E)] = jnp.transpose(t_ref[...])

    return pl.pallas_call(
        body,
        grid=(n_v,),
        in_specs=[pl.BlockSpec((E, _BVT), lambda v: (0, v))],
        out_specs=pl.BlockSpec((_BVT, 128), lambda v: (v, 0)),
        out_shape=jax.ShapeDtypeStruct((V, 128), jnp.float32),
        compiler_params=pltpu.CompilerParams(
            dimension_semantics=("arbitrary",)),
    )(embT)


def _pooled_sparsecore(idx3, emb_pad, B, CTX, E):
    n_chunks = idx3.shape[1]
    ep = emb_pad.shape[1]
    rows_pw = B // _NW
    idx_pw = rows_pw * CTX
    inv_ctx = jnp.float32(1.0 / CTX)
    mesh = plsc.VectorSubcoreMesh(core_axis_name="c", subcore_axis_name="s")

    @functools.partial(
        pl.kernel,
        mesh=mesh,
        out_type=jax.ShapeDtypeStruct((B, E), jnp.float32),
        scratch_types=[
            pltpu.VMEM((n_chunks, _CHUNK), jnp.int32),
            pltpu.VMEM((idx_pw, ep), jnp.float32),
            pltpu.VMEM((rows_pw, E), jnp.float32),
            pltpu.SemaphoreType.DMA,
        ],
    )
    def k(idx_hbm, table_hbm, out_hbm, idx_v, rows_v, pooled_v, sem):
        wid = lax.axis_index("s") * 2 + lax.axis_index("c")
        pltpu.sync_copy(idx_hbm.at[wid], idx_v)
        copies = [
            pltpu.make_async_copy(
                table_hbm.at[idx_v.at[c]],
                rows_v.at[pl.ds(c * _CHUNK, _CHUNK)],
                sem,
            )
            for c in range(n_chunks)
        ]
        for cp in copies:
            cp.start()
        for cp in copies:
            cp.wait()

        def body(r, carry):
            base = r * CTX
            for c4 in range(E // _LANES):
                sl = pl.ds(c4 * _LANES, _LANES)
                acc = rows_v[base, sl]
                for j in range(1, CTX):
                    acc = acc + rows_v[base + j, sl]
                pooled_v[r, sl] = acc * inv_ctx
            return carry

        lax.fori_loop(0, rows_pw, body, 0)
        pltpu.sync_copy(pooled_v, out_hbm.at[pl.ds(wid * rows_pw, rows_pw)])

    return k(idx3, emb_pad)


def _fused_tensorcore(pooled_bf, w, b2, B, V, E):
    n_v = -(-V // _BVF)

    def body(pooled_ref, w_ref, b_ref, o_ref, s_ref):
        p = pl.program_id(0)
        v = pl.program_id(1)

        @pl.when((p == 0) & (v == 0))
        def _():
            s_ref[...] = jnp.zeros_like(s_ref)

        x = lax.dot_general(
            w_ref[...].astype(jnp.bfloat16), pooled_ref[...],
            (((0,), (1,)), ((), ())),
            preferred_element_type=jnp.float32,
        )
        xb = x + b_ref[...].reshape(_BVF, 1)

        @pl.when(p == 0)
        def _():
            row = v * _BVF + lax.broadcasted_iota(jnp.int32, (_BVF, 1), 0)
            e = jnp.where(row < V, jnp.exp(xb), 0.0)
            s_ref[...] = s_ref[...] + jnp.sum(e, axis=0, keepdims=True)

        @pl.when(p == 1)
        def _():
            o_ref[...] = xb - jnp.log(s_ref[...])

    return pl.pallas_call(
        body,
        grid=(2, n_v),
        in_specs=[
            pl.BlockSpec((B, E), lambda p, v: (0, 0)),
            pl.BlockSpec((E, _BVF), lambda p, v: (0, v)),
            pl.BlockSpec((1, _BVF), lambda p, v: (0, v)),
        ],
        out_specs=pl.BlockSpec(
            (_BVF, B), lambda p, v: (jnp.where(p == 1, v, 0), 0)),
        out_shape=jax.ShapeDtypeStruct((V, B), jnp.float32),
        scratch_shapes=[pltpu.VMEM((1, B), jnp.float32)],
        compiler_params=pltpu.CompilerParams(
            dimension_semantics=("arbitrary", "arbitrary")),
    )(pooled_bf, w, b2)


def _lse_tensorcore(pooled_aug, w, b2, B, V, E):
    n_v = -(-V // _BV)

    def body(pooled_ref, w_ref, b_ref, lse_ref, s_ref):
        v = pl.program_id(0)

        @pl.when(v == 0)
        def _():
            s_ref[...] = jnp.zeros_like(s_ref)

        w_aug = jnp.concatenate(
            [w_ref[...], b_ref[...]], axis=0).astype(jnp.bfloat16)
        x = lax.dot_general(
            pooled_ref[...], w_aug,
            (((1,), (0,)), ((), ())),
            preferred_element_type=jnp.float32,
        )
        col = v * _BV + lax.broadcasted_iota(jnp.int32, (1, _BV), 1)
        e = jnp.where(col < V, jnp.exp2(x), 0.0)
        s_ref[...] = s_ref[...] + jnp.sum(e, axis=1, keepdims=True)

        @pl.when(v == n_v - 1)
        def _():
            lse_ref[...] = jnp.transpose(jnp.log(s_ref[...]))

    return pl.pallas_call(
        body,
        grid=(n_v,),
        in_specs=[
            pl.BlockSpec((B, E + 1), lambda v: (0, 0)),
            pl.BlockSpec((E, _BV), lambda v: (0, v)),
            pl.BlockSpec((1, _BV), lambda v: (0, v)),
        ],
        out_specs=pl.BlockSpec((1, B), lambda v: (0, 0)),
        out_shape=jax.ShapeDtypeStruct((1, B), jnp.float32),
        scratch_shapes=[pltpu.VMEM((B, 1), jnp.float32)],
        compiler_params=pltpu.CompilerParams(
            dimension_semantics=("arbitrary",)),
    )(pooled_aug, w, b2)


def _logprobs_tensorcore(pooled_aug, w, b2, lse_t, B, V, E):
    n_v = -(-V // _BVO)

    def body(pooled_ref, w_ref, b_ref, lse_ref, o_ref):
        w_aug = jnp.concatenate(
            [w_ref[...], b_ref[...]], axis=0).astype(jnp.bfloat16)
        x = lax.dot_general(
            w_aug, pooled_ref[...],
            (((0,), (1,)), ((), ())),
            preferred_element_type=jnp.float32,
        )
        o_ref[...] = x - lse_ref[...]

    return pl.pallas_call(
        body,
        grid=(n_v,),
        in_specs=[
            pl.BlockSpec((B, E + 1), lambda v: (0, 0)),
            pl.BlockSpec((E, _BVO), lambda v: (0, v)),
            pl.BlockSpec((1, _BVO), lambda v: (0, v)),
            pl.BlockSpec((1, B), lambda v: (0, 0)),
        ],
        out_specs=pl.BlockSpec((_BVO, B), lambda v: (v, 0)),
        out_shape=jax.ShapeDtypeStruct((V, B), jnp.float32),
        compiler_params=pltpu.CompilerParams(
            dimension_semantics=("arbitrary",)),
    )(pooled_aug, w, b2, lse_t)


def kernel(inputs, emb_table, W, b):
    B, CTX = inputs.shape
    V, E = W.shape
    idx_pw = B * CTX // _NW
    assert idx_pw % _CHUNK == 0 and B % _NW == 0 and E % _LANES == 0

    idx3 = inputs.astype(jnp.int32).reshape(_NW, idx_pw // _CHUNK, _CHUNK)
    emb_pad = _pad_table_tensorcore(emb_table.T, V, E)
    pooled = _pooled_sparsecore(idx3, emb_pad, B, CTX, E)

    log2e = 1.4426950408889634
    ones = jnp.ones((B, 1), jnp.float32)
    pooled_aug = jnp.concatenate(
        [pooled, ones], axis=1).astype(jnp.bfloat16)
    pooled_aug_s = jnp.concatenate(
        [pooled * log2e, ones * log2e], axis=1).astype(jnp.bfloat16)
    b2 = b.reshape(1, V)
    wt = W.T
    lse_t = _lse_tensorcore(pooled_aug_s, wt, b2, B, V, E)
    out_t = _logprobs_tensorcore(pooled_aug, wt, b2, lse_t, B, V, E)
    return out_t.T


# speedup vs baseline: 1.0071x; 1.0071x over previous
import functools

import jax
import jax.numpy as jnp
from jax import lax
from jax.experimental import pallas as pl
from jax.experimental.pallas import tpu as pltpu
from jax.experimental.pallas import tpu_sc as plsc

_NW = 32
_CHUNK = 128
_LANES = 16
_BV = 8192
_BVO = 4096
_BVT = 16384


def _pad_table_tensorcore(embT, V, E):
    n_v = -(-V // _BVT)

    def body(t_ref, o_ref):
        o_ref[:, pl.ds(0, E)] = jnp.transpose(t_ref[...])

    return pl.pallas_call(
        body,
        grid=(n_v,),
        in_specs=[pl.BlockSpec((E, _BVT), lambda v: (0, v))],
        out_specs=pl.BlockSpec((_BVT, 128), lambda v: (v, 0)),
        out_shape=jax.ShapeDtypeStruct((V, 128), jnp.float32),
        compiler_params=pltpu.CompilerParams(
            dimension_semantics=("arbitrary",)),
    )(embT)


def _pooled_sparsecore(idx3, emb_pad, B, CTX, E):
    n_chunks = idx3.shape[1]
    ep = emb_pad.shape[1]
    rows_pw = B // _NW
    idx_pw = rows_pw * CTX
    inv_ctx = jnp.float32(1.0 / CTX)
    mesh = plsc.VectorSubcoreMesh(core_axis_name="c", subcore_axis_name="s")

    @functools.partial(
        pl.kernel,
        mesh=mesh,
        out_type=jax.ShapeDtypeStruct((B, E), jnp.float32),
        scratch_types=[
            pltpu.VMEM((n_chunks, _CHUNK), jnp.int32),
            pltpu.VMEM((idx_pw, ep), jnp.float32),
            pltpu.VMEM((rows_pw, E), jnp.float32),
            pltpu.SemaphoreType.DMA,
        ],
    )
    def k(idx_hbm, table_hbm, out_hbm, idx_v, rows_v, pooled_v, sem):
        wid = lax.axis_index("s") * 2 + lax.axis_index("c")
        pltpu.sync_copy(idx_hbm.at[wid], idx_v)
        copies = [
            pltpu.make_async_copy(
                table_hbm.at[idx_v.at[c]],
                rows_v.at[pl.ds(c * _CHUNK, _CHUNK)],
                sem,
            )
            for c in range(n_chunks)
        ]
        for cp in copies:
            cp.start()
        for cp in copies:
            cp.wait()

        def body(r, carry):
            base = r * CTX
            for c4 in range(E // _LANES):
                sl = pl.ds(c4 * _LANES, _LANES)
                acc = rows_v[base, sl]
                for j in range(1, CTX):
                    acc = acc + rows_v[base + j, sl]
                pooled_v[r, sl] = acc * inv_ctx
            return carry

        lax.fori_loop(0, rows_pw, body, 0)
        pltpu.sync_copy(pooled_v, out_hbm.at[pl.ds(wid * rows_pw, rows_pw)])

    return k(idx3, emb_pad)


def _lse_tensorcore(pooled_aug, w, b2, B, V, E):
    n_v = -(-V // _BV)

    def body(pooled_ref, w_ref, b_ref, lse_ref, s_ref):
        v = pl.program_id(0)

        @pl.when(v == 0)
        def _():
            s_ref[...] = jnp.zeros_like(s_ref)

        w_aug = jnp.concatenate(
            [w_ref[...], b_ref[...]], axis=0).astype(jnp.bfloat16)
        x = lax.dot_general(
            pooled_ref[...], w_aug,
            (((1,), (0,)), ((), ())),
            preferred_element_type=jnp.float32,
        )
        col = v * _BV + lax.broadcasted_iota(jnp.int32, (1, _BV), 1)
        e = jnp.where(col < V, jnp.exp2(x), 0.0)
        s_ref[...] = s_ref[...] + jnp.sum(e, axis=1, keepdims=True)

        @pl.when(v == n_v - 1)
        def _():
            lse_ref[...] = jnp.transpose(jnp.log(s_ref[...]))

    return pl.pallas_call(
        body,
        grid=(n_v,),
        in_specs=[
            pl.BlockSpec((B, E + 1), lambda v: (0, 0)),
            pl.BlockSpec((E, _BV), lambda v: (0, v)),
            pl.BlockSpec((1, _BV), lambda v: (0, v)),
        ],
        out_specs=pl.BlockSpec((1, B), lambda v: (0, 0)),
        out_shape=jax.ShapeDtypeStruct((1, B), jnp.float32),
        scratch_shapes=[pltpu.VMEM((B, 1), jnp.float32)],
        compiler_params=pltpu.CompilerParams(
            dimension_semantics=("arbitrary",)),
    )(pooled_aug, w, b2)


def _logprobs_tensorcore(pooled_aug, w, b2, lse_t, B, V, E):
    n_v = -(-V // _BVO)

    def body(pooled_ref, w_ref, b_ref, lse_ref, o_ref):
        w_aug = jnp.concatenate(
            [w_ref[...], b_ref[...]], axis=0).astype(jnp.bfloat16)
        x = lax.dot_general(
            w_aug, pooled_ref[...],
            (((0,), (1,)), ((), ())),
            preferred_element_type=jnp.float32,
        )
        o_ref[...] = x - lse_ref[...]

    return pl.pallas_call(
        body,
        grid=(n_v,),
        in_specs=[
            pl.BlockSpec((B, E + 1), lambda v: (0, 0)),
            pl.BlockSpec((E, _BVO), lambda v: (0, v)),
            pl.BlockSpec((1, _BVO), lambda v: (0, v)),
            pl.BlockSpec((1, B), lambda v: (0, 0)),
        ],
        out_specs=pl.BlockSpec((_BVO, B), lambda v: (v, 0)),
        out_shape=jax.ShapeDtypeStruct((V, B), jnp.float32),
        compiler_params=pltpu.CompilerParams(
            dimension_semantics=("arbitrary",)),
    )(pooled_aug, w, b2, lse_t)


def kernel(inputs, emb_table, W, b):
    B, CTX = inputs.shape
    V, E = W.shape
    idx_pw = B * CTX // _NW
    assert idx_pw % _CHUNK == 0 and B % _NW == 0 and E % _LANES == 0

    idx3 = inputs.astype(jnp.int32).reshape(_NW, idx_pw // _CHUNK, _CHUNK)
    emb_pad = _pad_table_tensorcore(emb_table.T, V, E)
    pooled = _pooled_sparsecore(idx3, emb_pad, B, CTX, E)

    log2e = 1.4426950408889634
    ones = jnp.ones((B, 1), jnp.float32)
    pooled_aug = jnp.concatenate(
        [pooled, ones], axis=1).astype(jnp.bfloat16)
    pooled_aug_s = jnp.concatenate(
        [pooled * log2e, ones * log2e], axis=1).astype(jnp.bfloat16)
    b2 = b.reshape(1, V)
    wt = W.T
    lse_t = _lse_tensorcore(pooled_aug_s, wt, b2, B, V, E)
    out_t = _logprobs_tensorcore(pooled_aug, wt, b2, lse_t, B, V, E)
    return out_t.T


# speedup vs baseline: 1.0075x; 1.0003x over previous
import functools

import jax
import jax.numpy as jnp
from jax import lax
from jax.experimental import pallas as pl
from jax.experimental.pallas import tpu as pltpu
from jax.experimental.pallas import tpu_sc as plsc

_NW = 32
_CHUNK = 128
_LANES = 16
_BV = 8192
_BVO = 6144
_BVT = 16384


def _pad_table_tensorcore(embT, V, E):
    n_v = -(-V // _BVT)

    def body(t_ref, o_ref):
        o_ref[:, pl.ds(0, E)] = jnp.transpose(t_ref[...])

    return pl.pallas_call(
        body,
        grid=(n_v,),
        in_specs=[pl.BlockSpec((E, _BVT), lambda v: (0, v))],
        out_specs=pl.BlockSpec((_BVT, 128), lambda v: (v, 0)),
        out_shape=jax.ShapeDtypeStruct((V, 128), jnp.float32),
        compiler_params=pltpu.CompilerParams(
            dimension_semantics=("arbitrary",)),
    )(embT)


def _pooled_sparsecore(idx3, emb_pad, B, CTX, E):
    n_chunks = idx3.shape[1]
    ep = emb_pad.shape[1]
    rows_pw = B // _NW
    idx_pw = rows_pw * CTX
    inv_ctx = jnp.float32(1.0 / CTX)
    mesh = plsc.VectorSubcoreMesh(core_axis_name="c", subcore_axis_name="s")

    @functools.partial(
        pl.kernel,
        mesh=mesh,
        out_type=jax.ShapeDtypeStruct((B, E), jnp.float32),
        scratch_types=[
            pltpu.VMEM((n_chunks, _CHUNK), jnp.int32),
            pltpu.VMEM((idx_pw, ep), jnp.float32),
            pltpu.VMEM((rows_pw, E), jnp.float32),
            pltpu.SemaphoreType.DMA,
        ],
    )
    def k(idx_hbm, table_hbm, out_hbm, idx_v, rows_v, pooled_v, sem):
        wid = lax.axis_index("s") * 2 + lax.axis_index("c")
        pltpu.sync_copy(idx_hbm.at[wid], idx_v)
        copies = [
            pltpu.make_async_copy(
                table_hbm.at[idx_v.at[c]],
                rows_v.at[pl.ds(c * _CHUNK, _CHUNK)],
                sem,
            )
            for c in range(n_chunks)
        ]
        for cp in copies:
            cp.start()
        for cp in copies:
            cp.wait()

        def body(r, carry):
            base = r * CTX
            for c4 in range(E // _LANES):
                sl = pl.ds(c4 * _LANES, _LANES)
                acc = rows_v[base, sl]
                for j in range(1, CTX):
                    acc = acc + rows_v[base + j, sl]
                pooled_v[r, sl] = acc * inv_ctx
            return carry

        lax.fori_loop(0, rows_pw, body, 0)
        pltpu.sync_copy(pooled_v, out_hbm.at[pl.ds(wid * rows_pw, rows_pw)])

    return k(idx3, emb_pad)


def _lse_tensorcore(pooled_aug, w, b2, B, V, E):
    n_v = -(-V // _BV)

    def body(pooled_ref, w_ref, b_ref, lse_ref, s_ref):
        v = pl.program_id(0)

        @pl.when(v == 0)
        def _():
            s_ref[...] = jnp.zeros_like(s_ref)

        w_aug = jnp.concatenate(
            [w_ref[...], b_ref[...]], axis=0).astype(jnp.bfloat16)
        x = lax.dot_general(
            pooled_ref[...], w_aug,
            (((1,), (0,)), ((), ())),
            preferred_element_type=jnp.float32,
        )
        col = v * _BV + lax.broadcasted_iota(jnp.int32, (1, _BV), 1)
        e = jnp.where(col < V, jnp.exp2(x), 0.0)
        s_ref[...] = s_ref[...] + jnp.sum(e, axis=1, keepdims=True)

        @pl.when(v == n_v - 1)
        def _():
            lse_ref[...] = jnp.transpose(jnp.log(s_ref[...]))

    return pl.pallas_call(
        body,
        grid=(n_v,),
        in_specs=[
            pl.BlockSpec((B, E + 1), lambda v: (0, 0)),
            pl.BlockSpec((E, _BV), lambda v: (0, v)),
            pl.BlockSpec((1, _BV), lambda v: (0, v)),
        ],
        out_specs=pl.BlockSpec((1, B), lambda v: (0, 0)),
        out_shape=jax.ShapeDtypeStruct((1, B), jnp.float32),
        scratch_shapes=[pltpu.VMEM((B, 1), jnp.float32)],
        compiler_params=pltpu.CompilerParams(
            dimension_semantics=("arbitrary",)),
    )(pooled_aug, w, b2)


def _logprobs_tensorcore(pooled_aug, w, b2, lse_t, B, V, E):
    n_v = -(-V // _BVO)

    def body(pooled_ref, w_ref, b_ref, lse_ref, o_ref):
        w_aug = jnp.concatenate(
            [w_ref[...], b_ref[...]], axis=0).astype(jnp.bfloat16)
        x = lax.dot_general(
            w_aug, pooled_ref[...],
            (((0,), (1,)), ((), ())),
            preferred_element_type=jnp.float32,
        )
        o_ref[...] = x - lse_ref[...]

    return pl.pallas_call(
        body,
        grid=(n_v,),
        in_specs=[
            pl.BlockSpec((B, E + 1), lambda v: (0, 0)),
            pl.BlockSpec((E, _BVO), lambda v: (0, v)),
            pl.BlockSpec((1, _BVO), lambda v: (0, v)),
            pl.BlockSpec((1, B), lambda v: (0, 0)),
        ],
        out_specs=pl.BlockSpec((_BVO, B), lambda v: (v, 0)),
        out_shape=jax.ShapeDtypeStruct((V, B), jnp.float32),
        compiler_params=pltpu.CompilerParams(
            dimension_semantics=("arbitrary",)),
    )(pooled_aug, w, b2, lse_t)


def kernel(inputs, emb_table, W, b):
    B, CTX = inputs.shape
    V, E = W.shape
    idx_pw = B * CTX // _NW
    assert idx_pw % _CHUNK == 0 and B % _NW == 0 and E % _LANES == 0

    idx3 = inputs.astype(jnp.int32).reshape(_NW, idx_pw // _CHUNK, _CHUNK)
    emb_pad = _pad_table_tensorcore(emb_table.T, V, E)
    pooled = _pooled_sparsecore(idx3, emb_pad, B, CTX, E)

    log2e = 1.4426950408889634
    ones = jnp.ones((B, 1), jnp.float32)
    pooled_aug = jnp.concatenate(
        [pooled, ones], axis=1).astype(jnp.bfloat16)
    pooled_aug_s = jnp.concatenate(
        [pooled * log2e, ones * log2e], axis=1).astype(jnp.bfloat16)
    b2 = b.reshape(1, V)
    wt = W.T
    lse_t = _lse_tensorcore(pooled_aug_s, wt, b2, B, V, E)
    out_t = _logprobs_tensorcore(pooled_aug, wt, b2, lse_t, B, V, E)
    return out_t.T


# speedup vs baseline: 1.0075x; 1.0000x over previous
import functools

import jax
import jax.numpy as jnp
from jax import lax
from jax.experimental import pallas as pl
from jax.experimental.pallas import tpu as pltpu
from jax.experimental.pallas import tpu_sc as plsc

_NW = 32
_CHUNK = 128
_LANES = 16
_BV = 8192
_BVO = 4096
_BVT = 16384


def _pad_table_tensorcore(embT, V, E):
    n_v = -(-V // _BVT)

    def body(t_ref, o_ref):
        o_ref[:, pl.ds(0, E)] = jnp.transpose(t_ref[...])

    return pl.pallas_call(
        body,
        grid=(n_v,),
        in_specs=[pl.BlockSpec((E, _BVT), lambda v: (0, v))],
        out_specs=pl.BlockSpec((_BVT, 128), lambda v: (v, 0)),
        out_shape=jax.ShapeDtypeStruct((V, 128), jnp.float32),
        compiler_params=pltpu.CompilerParams(
            dimension_semantics=("arbitrary",)),
    )(embT)


def _pooled_sparsecore(idx3, emb_pad, B, CTX, E):
    n_chunks = idx3.shape[1]
    ep = emb_pad.shape[1]
    rows_pw = B // _NW
    idx_pw = rows_pw * CTX
    inv_ctx = jnp.float32(1.0 / CTX)
    mesh = plsc.VectorSubcoreMesh(core_axis_name="c", subcore_axis_name="s")

    @functools.partial(
        pl.kernel,
        mesh=mesh,
        out_type=jax.ShapeDtypeStruct((B, E), jnp.float32),
        scratch_types=[
            pltpu.VMEM((n_chunks, _CHUNK), jnp.int32),
            pltpu.VMEM((idx_pw, ep), jnp.float32),
            pltpu.VMEM((rows_pw, E), jnp.float32),
            pltpu.SemaphoreType.DMA,
        ],
    )
    def k(idx_hbm, table_hbm, out_hbm, idx_v, rows_v, pooled_v, sem):
        wid = lax.axis_index("s") * 2 + lax.axis_index("c")
        pltpu.sync_copy(idx_hbm.at[wid], idx_v)
        copies = [
            pltpu.make_async_copy(
                table_hbm.at[idx_v.at[c]],
                rows_v.at[pl.ds(c * _CHUNK, _CHUNK)],
                sem,
            )
            for c in range(n_chunks)
        ]
        for cp in copies:
            cp.start()
        for cp in copies:
            cp.wait()

        def body(r, carry):
            base = r * CTX
            for c4 in range(E // _LANES):
                sl = pl.ds(c4 * _LANES, _LANES)
                acc = rows_v[base, sl]
                for j in range(1, CTX):
                    acc = acc + rows_v[base + j, sl]
                pooled_v[r, sl] = acc * inv_ctx
            return carry

        lax.fori_loop(0, rows_pw, body, 0)
        pltpu.sync_copy(pooled_v, out_hbm.at[pl.ds(wid * rows_pw, rows_pw)])

    return k(idx3, emb_pad)


def _lse_tensorcore(pooled_aug, w, b2, B, V, E):
    n_v = -(-V // _BV)

    def body(pooled_ref, w_ref, b_ref, lse_ref, s_ref):
        v = pl.program_id(0)

        @pl.when(v == 0)
        def _():
            s_ref[...] = jnp.zeros_like(s_ref)

        w_aug = jnp.concatenate(
            [w_ref[...], b_ref[...]], axis=0).astype(jnp.bfloat16)
        x = lax.dot_general(
            pooled_ref[...], w_aug,
            (((1,), (0,)), ((), ())),
            preferred_element_type=jnp.float32,
        )
        col = v * _BV + lax.broadcasted_iota(jnp.int32, (1, _BV), 1)
        e = jnp.where(col < V, jnp.exp2(x), 0.0)
        s_ref[...] = s_ref[...] + jnp.sum(e, axis=1, keepdims=True)

        @pl.when(v == n_v - 1)
        def _():
            lse_ref[...] = jnp.transpose(jnp.log(s_ref[...]))

    return pl.pallas_call(
        body,
        grid=(n_v,),
        in_specs=[
            pl.BlockSpec((B, E + 1), lambda v: (0, 0)),
            pl.BlockSpec((E, _BV), lambda v: (0, v)),
            pl.BlockSpec((1, _BV), lambda v: (0, v)),
        ],
        out_specs=pl.BlockSpec((1, B), lambda v: (0, 0)),
        out_shape=jax.ShapeDtypeStruct((1, B), jnp.float32),
        scratch_shapes=[pltpu.VMEM((B, 1), jnp.float32)],
        compiler_params=pltpu.CompilerParams(
            dimension_semantics=("arbitrary",)),
    )(pooled_aug, w, b2)


def _logprobs_tensorcore(pooled_aug, w, b2, lse_t, B, V, E):
    n_v = -(-V // _BVO)

    def body(pooled_ref, w_ref, b_ref, lse_ref, o_ref):
        w_aug = jnp.concatenate(
            [w_ref[...], b_ref[...]], axis=0).astype(jnp.bfloat16)
        x = lax.dot_general(
            w_aug, pooled_ref[...],
            (((0,), (1,)), ((), ())),
            preferred_element_type=jnp.float32,
        )
        o_ref[...] = x - lse_ref[...]

    return pl.pallas_call(
        body,
        grid=(n_v,),
        in_specs=[
            pl.BlockSpec((B, E + 1), lambda v: (0, 0)),
            pl.BlockSpec((E, _BVO), lambda v: (0, v)),
            pl.BlockSpec((1, _BVO), lambda v: (0, v)),
            pl.BlockSpec((1, B), lambda v: (0, 0)),
        ],
        out_specs=pl.BlockSpec((_BVO, B), lambda v: (v, 0)),
        out_shape=jax.ShapeDtypeStruct((V, B), jnp.float32),
        compiler_params=pltpu.CompilerParams(
            dimension_semantics=("arbitrary",)),
    )(pooled_aug, w, b2, lse_t)


def kernel(inputs, emb_table, W, b):
    B, CTX = inputs.shape
    V, E = W.shape
    idx_pw = B * CTX // _NW
    assert idx_pw % _CHUNK == 0 and B % _NW == 0 and E % _LANES == 0

    idx3 = inputs.astype(jnp.int32).reshape(_NW, idx_pw // _CHUNK, _CHUNK)
    emb_pad = _pad_table_tensorcore(emb_table.T, V, E)
    pooled = _pooled_sparsecore(idx3, emb_pad, B, CTX, E)

    log2e = 1.4426950408889634
    ones = jnp.ones((B, 1), jnp.float32)
    pooled_aug = jnp.concatenate(
        [pooled, ones], axis=1).astype(jnp.bfloat16)
    pooled_aug_s = jnp.concatenate(
        [pooled * log2e, ones * log2e], axis=1).astype(jnp.bfloat16)
    b2 = b.reshape(1, V)
    wt = W.T
    lse_t = _lse_tensorcore(pooled_aug_s, wt, b2, B, V, E)
    out_t = _logprobs_tensorcore(pooled_aug, wt, b2, lse_t, B, V, E)
    return out_t.T
